# baseline (speedup 1.0000x reference)
import functools

import jax
import jax.numpy as jnp
from jax.experimental import pallas as pl
from jax.experimental.pallas import tpu as pltpu

_BM = 512


def _gcn_body(adj_d_ref, adj_p_ref, h_d_ref, h_p_ref, wt_ref, b_ref, a_ref,
              wfct_ref, bfc_ref, e_ref, s_ref, fts_ref):
    sdx = pl.program_id(0)
    i = pl.program_id(2)

    def work(adj_ref, h_ref):
        @pl.when(i == 0)
        def _fts():
            fts_ref[...] = jnp.dot(
                h_ref[...].astype(jnp.bfloat16), wt_ref[0, 0],
                preferred_element_type=jnp.float32).astype(jnp.bfloat16)

        adj = adj_ref[0].astype(jnp.bfloat16)
        acc = jnp.dot(adj, fts_ref[...], preferred_element_type=jnp.float32)
        out = acc + b_ref[0, 0]
        out = jnp.where(out >= 0, out, a_ref[0, 0] * out)
        e_ref[0, 0] = out.astype(jnp.bfloat16)
        pre = jnp.dot(out.astype(jnp.bfloat16), wfct_ref[0],
                      preferred_element_type=jnp.float32) + bfc_ref[0]
        col = jnp.sum(jnp.tanh(pre), axis=0, keepdims=True)

        @pl.when(i == 0)
        def _init():
            s_ref[0, 0] = col

        @pl.when(i > 0)
        def _acc():
            s_ref[0, 0] = s_ref[0, 0] + col

    @pl.when(sdx == 0)
    def _d():
        work(adj_d_ref, h_d_ref)

    @pl.when(sdx == 1)
    def _p():
        work(adj_p_ref, h_p_ref)


def _combine_body(e_ref, beta_ref, zd_ref, zp_ref, *, num_mp):
    sdx = pl.program_id(0)

    def mix():
        z = e_ref[0, 0].astype(jnp.float32) * beta_ref[0, 0]
        for p in range(1, num_mp):
            z = z + e_ref[0, p].astype(jnp.float32) * beta_ref[0, p]
        return z

    @pl.when(sdx == 0)
    def _d():
        zd_ref[...] = mix()

    @pl.when(sdx == 1)
    def _p():
        zp_ref[...] = mix()


def kernel(h_d, h_p, mps_d, mps_p, W_dg, b_dg, a_dg, W_pt, b_pt, a_pt,
           Wfc_d, bfc_d, att_d, Wfc_p, bfc_p, att_p):
    P, N, _ = mps_d.shape
    D = h_d.shape[1]
    nb = N // _BM

    Wt = jnp.stack([jnp.transpose(W_dg, (0, 2, 1)),
                    jnp.transpose(W_pt, (0, 2, 1))]).astype(jnp.bfloat16)
    b4 = jnp.stack([b_dg, b_pt]).reshape(2, P, 1, D)
    a4 = jnp.broadcast_to(jnp.stack([a_dg, a_pt]).reshape(2, P, 1, 1),
                          (2, P, 1, D))
    wfct = jnp.stack([Wfc_d.T, Wfc_p.T]).astype(jnp.bfloat16)
    bfc3 = jnp.stack([bfc_d, bfc_p]).reshape(2, 1, D)
    att3 = jnp.stack([att_d, att_p])

    e, stat = pl.pallas_call(
        _gcn_body,
        grid=(2, P, nb),
        in_specs=[
            pl.BlockSpec((1, _BM, N),
                         lambda s, p, i: (jnp.where(s == 0, p, P - 1),
                                          jnp.where(s == 0, i, nb - 1), 0)),
            pl.BlockSpec((1, _BM, N),
                         lambda s, p, i: (jnp.where(s == 1, p, 0),
                                          jnp.where(s == 1, i, 0), 0)),
            pl.BlockSpec((N, D), lambda s, p, i: (0, 0)),
            pl.BlockSpec((N, D), lambda s, p, i: (0, 0)),
            pl.BlockSpec((1, 1, D, D), lambda s, p, i: (s, p, 0, 0)),
            pl.BlockSpec((1, 1, 1, D), lambda s, p, i: (s, p, 0, 0)),
            pl.BlockSpec((1, 1, 1, D), lambda s, p, i: (s, p, 0, 0)),
            pl.BlockSpec((1, D, D), lambda s, p, i: (s, 0, 0)),
            pl.BlockSpec((1, 1, D), lambda s, p, i: (s, 0, 0)),
        ],
        out_specs=[
            pl.BlockSpec((1, 1, _BM, D), lambda s, p, i: (s, p, i, 0)),
            pl.BlockSpec((1, 1, 1, D), lambda s, p, i: (s, p, 0, 0)),
        ],
        out_shape=[
            jax.ShapeDtypeStruct((2, P, N, D), jnp.bfloat16),
            jax.ShapeDtypeStruct((2, P, 1, D), jnp.float32),
        ],
        scratch_shapes=[pltpu.VMEM((N, D), jnp.bfloat16)],
    )(mps_d, mps_p, h_d, h_p, Wt, b4, a4, wfct, bfc3)

    mean_t = stat[:, :, 0, :] / jnp.float32(N)
    logits = jnp.sum(mean_t * att3, axis=2)
    beta = jax.nn.softmax(logits, axis=1)
    beta4 = jnp.broadcast_to(beta.reshape(2, P, 1, 1),
                             (2, P, 1, D)).astype(jnp.float32)

    z_d, z_p = pl.pallas_call(
        functools.partial(_combine_body, num_mp=P),
        grid=(2, nb),
        in_specs=[
            pl.BlockSpec((1, P, _BM, D), lambda s, i: (s, 0, i, 0)),
            pl.BlockSpec((1, P, 1, D), lambda s, i: (s, 0, 0, 0)),
        ],
        out_specs=[
            pl.BlockSpec((_BM, D),
                         lambda s, i: (jnp.where(s == 0, i, nb - 1), 0)),
            pl.BlockSpec((_BM, D),
                         lambda s, i: (jnp.where(s == 1, i, 0), 0)),
        ],
        out_shape=[
            jax.ShapeDtypeStruct((N, D), jnp.float32),
            jax.ShapeDtypeStruct((N, D), jnp.float32),
        ],
    )(e, beta4)
    return (z_d, z_p)


# speedup vs baseline: 1.0175x; 1.0175x over previous
import functools

import jax
import jax.numpy as jnp
from jax.experimental import pallas as pl
from jax.experimental.pallas import tpu as pltpu

_BM = 512


def _fts_body(h_d_ref, h_p_ref, wt_ref, fts_ref):
    sdx = pl.program_id(0)

    @pl.when(sdx == 0)
    def _d():
        fts = jnp.dot(h_d_ref[...], wt_ref[0, 0],
                      preferred_element_type=jnp.float32)
        fts_ref[0, 0] = fts.astype(jnp.bfloat16)

    @pl.when(sdx == 1)
    def _p():
        fts = jnp.dot(h_p_ref[...], wt_ref[0, 0],
                      preferred_element_type=jnp.float32)
        fts_ref[0, 0] = fts.astype(jnp.bfloat16)


def _gcn_body(adj_d_ref, adj_p_ref, fts_ref, b_ref, a_ref, wfct_ref,
              bfc_ref, e_ref, s_ref):
    sdx = pl.program_id(0)
    i = pl.program_id(2)

    def work(adj_ref):
        adj = adj_ref[0].astype(jnp.bfloat16)
        acc = jnp.dot(adj, fts_ref[0, 0], preferred_element_type=jnp.float32)
        out = acc + b_ref[0, 0]
        out = jnp.where(out >= 0, out, a_ref[0, 0] * out)
        e_ref[0, 0] = out.astype(jnp.bfloat16)
        pre = jnp.dot(out.astype(jnp.bfloat16), wfct_ref[0],
                      preferred_element_type=jnp.float32) + bfc_ref[0]
        col = jnp.sum(jnp.tanh(pre), axis=0, keepdims=True)

        @pl.when(i == 0)
        def _init():
            s_ref[0, 0] = col

        @pl.when(i > 0)
        def _acc():
            s_ref[0, 0] = s_ref[0, 0] + col

    @pl.when(sdx == 0)
    def _d():
        work(adj_d_ref)

    @pl.when(sdx == 1)
    def _p():
        work(adj_p_ref)


def _combine_body(e_ref, beta_ref, zd_ref, zp_ref, *, num_mp):
    sdx = pl.program_id(0)

    def mix():
        z = e_ref[0, 0].astype(jnp.float32) * beta_ref[0, 0]
        for p in range(1, num_mp):
            z = z + e_ref[0, p].astype(jnp.float32) * beta_ref[0, p]
        return z

    @pl.when(sdx == 0)
    def _d():
        zd_ref[...] = mix()

    @pl.when(sdx == 1)
    def _p():
        zp_ref[...] = mix()


def kernel(h_d, h_p, mps_d, mps_p, W_dg, b_dg, a_dg, W_pt, b_pt, a_pt,
           Wfc_d, bfc_d, att_d, Wfc_p, bfc_p, att_p):
    P, N, _ = mps_d.shape
    D = h_d.shape[1]
    nb = N // _BM

    Wt = jnp.stack([jnp.transpose(W_dg, (0, 2, 1)),
                    jnp.transpose(W_pt, (0, 2, 1))])
    b4 = jnp.stack([b_dg, b_pt]).reshape(2, P, 1, D)
    a4 = jnp.broadcast_to(jnp.stack([a_dg, a_pt]).reshape(2, P, 1, 1),
                          (2, P, 1, D))
    wfct = jnp.stack([Wfc_d.T, Wfc_p.T]).astype(jnp.bfloat16)
    bfc3 = jnp.stack([bfc_d, bfc_p]).reshape(2, 1, D)
    att3 = jnp.stack([att_d, att_p])

    fts = pl.pallas_call(
        _fts_body,
        grid=(2, P),
        in_specs=[
            pl.BlockSpec((N, D), lambda s, p: (0, 0)),
            pl.BlockSpec((N, D), lambda s, p: (0, 0)),
            pl.BlockSpec((1, 1, D, D), lambda s, p: (s, p, 0, 0)),
        ],
        out_specs=pl.BlockSpec((1, 1, N, D), lambda s, p: (s, p, 0, 0)),
        out_shape=jax.ShapeDtypeStruct((2, P, N, D), jnp.bfloat16),
    )(h_d, h_p, Wt)

    e, stat = pl.pallas_call(
        _gcn_body,
        grid=(2, P, nb),
        in_specs=[
            pl.BlockSpec((1, _BM, N),
                         lambda s, p, i: (jnp.where(s == 0, p, P - 1),
                                          jnp.where(s == 0, i, nb - 1), 0)),
            pl.BlockSpec((1, _BM, N),
                         lambda s, p, i: (jnp.where(s == 1, p, 0),
                                          jnp.where(s == 1, i, 0), 0)),
            pl.BlockSpec((1, 1, N, D), lambda s, p, i: (s, p, 0, 0)),
            pl.BlockSpec((1, 1, 1, D), lambda s, p, i: (s, p, 0, 0)),
            pl.BlockSpec((1, 1, 1, D), lambda s, p, i: (s, p, 0, 0)),
            pl.BlockSpec((1, D, D), lambda s, p, i: (s, 0, 0)),
            pl.BlockSpec((1, 1, D), lambda s, p, i: (s, 0, 0)),
        ],
        out_specs=[
            pl.BlockSpec((1, 1, _BM, D), lambda s, p, i: (s, p, i, 0)),
            pl.BlockSpec((1, 1, 1, D), lambda s, p, i: (s, p, 0, 0)),
        ],
        out_shape=[
            jax.ShapeDtypeStruct((2, P, N, D), jnp.bfloat16),
            jax.ShapeDtypeStruct((2, P, 1, D), jnp.float32),
        ],
        compiler_params=pltpu.CompilerParams(
            dimension_semantics=("parallel", "arbitrary", "arbitrary")),
    )(mps_d, mps_p, fts, b4, a4, wfct, bfc3)

    mean_t = stat[:, :, 0, :] / jnp.float32(N)
    logits = jnp.sum(mean_t * att3, axis=2)
    beta = jax.nn.softmax(logits, axis=1)
    beta4 = jnp.broadcast_to(beta.reshape(2, P, 1, 1),
                             (2, P, 1, D)).astype(jnp.float32)

    z_d, z_p = pl.pallas_call(
        functools.partial(_combine_body, num_mp=P),
        grid=(2, nb),
        in_specs=[
            pl.BlockSpec((1, P, _BM, D), lambda s, i: (s, 0, i, 0)),
            pl.BlockSpec((1, P, 1, D), lambda s, i: (s, 0, 0, 0)),
        ],
        out_specs=[
            pl.BlockSpec((_BM, D),
                         lambda s, i: (jnp.where(s == 0, i, nb - 1), 0)),
            pl.BlockSpec((_BM, D),
                         lambda s, i: (jnp.where(s == 1, i, 0), 0)),
        ],
        out_shape=[
            jax.ShapeDtypeStruct((N, D), jnp.float32),
            jax.ShapeDtypeStruct((N, D), jnp.float32),
        ],
    )(e, beta4)
    return (z_d, z_p)


# speedup vs baseline: 1.0280x; 1.0104x over previous
import jax
import jax.numpy as jnp
from jax.experimental import pallas as pl
from jax.experimental.pallas import tpu as pltpu

_BM = 512


def _fts_body(h_d_ref, h_p_ref, wt_ref, fts_ref):
    sdx = pl.program_id(0)

    @pl.when(sdx == 0)
    def _d():
        fts = jnp.dot(h_d_ref[...], wt_ref[0, 0],
                      preferred_element_type=jnp.float32)
        fts_ref[0, 0] = fts.astype(jnp.bfloat16)

    @pl.when(sdx == 1)
    def _p():
        fts = jnp.dot(h_p_ref[...], wt_ref[0, 0],
                      preferred_element_type=jnp.float32)
        fts_ref[0, 0] = fts.astype(jnp.bfloat16)


def _gcn_body(num_mp, inv_n,
              adj_d_ref, adj_p_ref, fts_ref, b_ref, a_ref, wfct_ref,
              bfc_ref, att_ref, zd_ref, zp_ref, e_scr, st_scr):
    sdx = pl.program_id(0)
    p = pl.program_id(1)
    i = pl.program_id(2)
    bm = adj_d_ref.shape[1]

    def work(adj_ref):
        adj = adj_ref[0].astype(jnp.bfloat16)
        acc = jnp.dot(adj, fts_ref[0, 0], preferred_element_type=jnp.float32)
        out = acc + b_ref[0, 0]
        out = jnp.where(out >= 0, out, a_ref[0, 0] * out)
        e_scr[p, pl.ds(i * bm, bm), :] = out.astype(jnp.bfloat16)
        pre = jnp.dot(out.astype(jnp.bfloat16), wfct_ref[0],
                      preferred_element_type=jnp.float32) + bfc_ref[0]
        col = jnp.sum(jnp.tanh(pre), axis=0, keepdims=True)

        @pl.when(i == 0)
        def _init():
            st_scr[p] = col

        @pl.when(i > 0)
        def _acc():
            st_scr[p] = st_scr[p] + col

    @pl.when((sdx == 0) & (p < num_mp))
    def _d():
        work(adj_d_ref)

    @pl.when((sdx == 1) & (p < num_mp))
    def _p():
        work(adj_p_ref)

    @pl.when(p == num_mp)
    def _combine():
        att = att_ref[0]
        ls = [jnp.sum(st_scr[q] * att, keepdims=True) * inv_n
              for q in range(num_mp)]
        m = ls[0]
        for q in range(1, num_mp):
            m = jnp.maximum(m, ls[q])
        ws = [jnp.exp(l - m) for l in ls]
        den = ws[0]
        for q in range(1, num_mp):
            den = den + ws[q]
        z = None
        for q in range(num_mp):
            blk = e_scr[q, pl.ds(i * bm, bm), :].astype(jnp.float32)
            term = blk * (ws[q] / den)
            z = term if z is None else z + term

        @pl.when(sdx == 0)
        def _zd():
            zd_ref[...] = z

        @pl.when(sdx == 1)
        def _zp():
            zp_ref[...] = z


def kernel(h_d, h_p, mps_d, mps_p, W_dg, b_dg, a_dg, W_pt, b_pt, a_pt,
           Wfc_d, bfc_d, att_d, Wfc_p, bfc_p, att_p):
    P, N, _ = mps_d.shape
    D = h_d.shape[1]
    nb = N // _BM

    Wt = jnp.stack([jnp.transpose(W_dg, (0, 2, 1)),
                    jnp.transpose(W_pt, (0, 2, 1))])
    b4 = jnp.stack([b_dg, b_pt]).reshape(2, P, 1, D)
    a4 = jnp.broadcast_to(jnp.stack([a_dg, a_pt]).reshape(2, P, 1, 1),
                          (2, P, 1, D))
    wfct = jnp.stack([Wfc_d.T, Wfc_p.T]).astype(jnp.bfloat16)
    bfc3 = jnp.stack([bfc_d, bfc_p]).reshape(2, 1, D)
    att3 = jnp.stack([att_d, att_p])

    fts = pl.pallas_call(
        _fts_body,
        grid=(2, P),
        in_specs=[
            pl.BlockSpec((N, D), lambda s, p: (0, 0)),
            pl.BlockSpec((N, D), lambda s, p: (0, 0)),
            pl.BlockSpec((1, 1, D, D), lambda s, p: (s, p, 0, 0)),
        ],
        out_specs=pl.BlockSpec((1, 1, N, D), lambda s, p: (s, p, 0, 0)),
        out_shape=jax.ShapeDtypeStruct((2, P, N, D), jnp.bfloat16),
    )(h_d, h_p, Wt)

    def adj_d_map(s, p, i):
        act = (s == 0) & (p < P)
        return (jnp.where(act, p, P - 1), jnp.where(act, i, nb - 1), 0)

    def adj_p_map(s, p, i):
        return (jnp.where(s == 0, 0, jnp.where(p < P, p, P - 1)),
                jnp.where(s == 0, 0, jnp.where(p < P, i, nb - 1)), 0)

    def zd_map(s, p, i):
        return (jnp.where(s == 0, jnp.where(p < P, 0, i), nb - 1), 0)

    def zp_map(s, p, i):
        return (jnp.where((s == 1) & (p == P), i, 0), 0)

    z_d, z_p = pl.pallas_call(
        lambda *refs: _gcn_body(P, 1.0 / N, *refs),
        grid=(2, P + 1, nb),
        in_specs=[
            pl.BlockSpec((1, _BM, N), adj_d_map),
            pl.BlockSpec((1, _BM, N), adj_p_map),
            pl.BlockSpec((1, 1, N, D),
                         lambda s, p, i: (s, jnp.minimum(p, P - 1), 0, 0)),
            pl.BlockSpec((1, 1, 1, D),
                         lambda s, p, i: (s, jnp.minimum(p, P - 1), 0, 0)),
            pl.BlockSpec((1, 1, 1, D),
                         lambda s, p, i: (s, jnp.minimum(p, P - 1), 0, 0)),
            pl.BlockSpec((1, D, D), lambda s, p, i: (s, 0, 0)),
            pl.BlockSpec((1, 1, D), lambda s, p, i: (s, 0, 0)),
            pl.BlockSpec((1, 1, D), lambda s, p, i: (s, 0, 0)),
        ],
        out_specs=[
            pl.BlockSpec((_BM, D), zd_map),
            pl.BlockSpec((_BM, D), zp_map),
        ],
        out_shape=[
            jax.ShapeDtypeStruct((N, D), jnp.float32),
            jax.ShapeDtypeStruct((N, D), jnp.float32),
        ],
        scratch_shapes=[
            pltpu.VMEM((P, N, D), jnp.bfloat16),
            pltpu.VMEM((P, 1, D), jnp.float32),
        ],
    )(mps_d, mps_p, fts, b4, a4, wfct, bfc3, att3)
    return (z_d, z_p)


# speedup vs baseline: 1.0541x; 1.0254x over previous
import jax
import jax.numpy as jnp
from jax.experimental import pallas as pl
from jax.experimental.pallas import tpu as pltpu

_BM = 512


def _gcn_body(num_mp, inv_n,
              adj_d_ref, adj_p_ref, h_d_ref, h_p_ref, wt_ref, b_ref, a_ref,
              wfct_ref, bfc_ref, att_ref, zd_ref, zp_ref,
              fts_scr, e_scr, st_scr):
    sdx = pl.program_id(0)
    p = pl.program_id(1)
    i = pl.program_id(2)
    bm = adj_d_ref.shape[1]

    def work(adj_ref, h_ref):
        @pl.when(p == 0)
        def _prelude():
            for q in range(num_mp):
                @pl.when(i == q)
                def _fts(q=q):
                    fts = jnp.dot(h_ref[...].astype(jnp.bfloat16),
                                  wt_ref[0, q],
                                  preferred_element_type=jnp.float32)
                    fts_scr[q] = fts.astype(jnp.bfloat16)

        adj = adj_ref[0].astype(jnp.bfloat16)
        acc = jnp.dot(adj, fts_scr[p], preferred_element_type=jnp.float32)
        out = acc + b_ref[0, 0]
        out = jnp.where(out >= 0, out, a_ref[0, 0] * out)
        e_scr[p, pl.ds(i * bm, bm), :] = out.astype(jnp.bfloat16)
        pre = jnp.dot(out.astype(jnp.bfloat16), wfct_ref[0],
                      preferred_element_type=jnp.float32) + bfc_ref[0]
        col = jnp.sum(jnp.tanh(pre), axis=0, keepdims=True)

        @pl.when(i == 0)
        def _init():
            st_scr[p] = col

        @pl.when(i > 0)
        def _acc():
            st_scr[p] = st_scr[p] + col

    @pl.when((sdx == 0) & (p < num_mp))
    def _d():
        work(adj_d_ref, h_d_ref)

    @pl.when((sdx == 1) & (p < num_mp))
    def _p():
        work(adj_p_ref, h_p_ref)

    @pl.when(p == num_mp)
    def _combine():
        att = att_ref[0]
        ls = [jnp.sum(st_scr[q] * att, keepdims=True) * inv_n
              for q in range(num_mp)]
        m = ls[0]
        for q in range(1, num_mp):
            m = jnp.maximum(m, ls[q])
        ws = [jnp.exp(l - m) for l in ls]
        den = ws[0]
        for q in range(1, num_mp):
            den = den + ws[q]
        z = None
        for q in range(num_mp):
            blk = e_scr[q, pl.ds(i * bm, bm), :].astype(jnp.float32)
            term = blk * (ws[q] / den)
            z = term if z is None else z + term

        @pl.when(sdx == 0)
        def _zd():
            zd_ref[...] = z

        @pl.when(sdx == 1)
        def _zp():
            zp_ref[...] = z


def kernel(h_d, h_p, mps_d, mps_p, W_dg, b_dg, a_dg, W_pt, b_pt, a_pt,
           Wfc_d, bfc_d, att_d, Wfc_p, bfc_p, att_p):
    P, N, _ = mps_d.shape
    D = h_d.shape[1]
    nb = N // _BM

    Wt = jnp.stack([jnp.transpose(W_dg, (0, 2, 1)),
                    jnp.transpose(W_pt, (0, 2, 1))]).astype(jnp.bfloat16)
    b4 = jnp.stack([b_dg, b_pt]).reshape(2, P, 1, D)
    a4 = jnp.broadcast_to(jnp.stack([a_dg, a_pt]).reshape(2, P, 1, 1),
                          (2, P, 1, D))
    wfct = jnp.stack([Wfc_d.T, Wfc_p.T]).astype(jnp.bfloat16)
    bfc3 = jnp.stack([bfc_d, bfc_p]).reshape(2, 1, D)
    att3 = jnp.stack([att_d, att_p])

    def adj_d_map(s, p, i):
        act = (s == 0) & (p < P)
        return (jnp.where(act, p, P - 1), jnp.where(act, i, nb - 1), 0)

    def adj_p_map(s, p, i):
        return (jnp.where(s == 0, 0, jnp.where(p < P, p, P - 1)),
                jnp.where(s == 0, 0, jnp.where(p < P, i, nb - 1)), 0)

    def zd_map(s, p, i):
        return (jnp.where(s == 0, jnp.where(p < P, 0, i), nb - 1), 0)

    def zp_map(s, p, i):
        return (jnp.where((s == 1) & (p == P), i, 0), 0)

    z_d, z_p = pl.pallas_call(
        lambda *refs: _gcn_body(P, 1.0 / N, *refs),
        grid=(2, P + 1, nb),
        in_specs=[
            pl.BlockSpec((1, _BM, N), adj_d_map),
            pl.BlockSpec((1, _BM, N), adj_p_map),
            pl.BlockSpec((N, D), lambda s, p, i: (0, 0)),
            pl.BlockSpec((N, D), lambda s, p, i: (0, 0)),
            pl.BlockSpec((1, P, D, D), lambda s, p, i: (s, 0, 0, 0)),
            pl.BlockSpec((1, 1, 1, D),
                         lambda s, p, i: (s, jnp.minimum(p, P - 1), 0, 0)),
            pl.BlockSpec((1, 1, 1, D),
                         lambda s, p, i: (s, jnp.minimum(p, P - 1), 0, 0)),
            pl.BlockSpec((1, D, D), lambda s, p, i: (s, 0, 0)),
            pl.BlockSpec((1, 1, D), lambda s, p, i: (s, 0, 0)),
            pl.BlockSpec((1, 1, D), lambda s, p, i: (s, 0, 0)),
        ],
        out_specs=[
            pl.BlockSpec((_BM, D), zd_map),
            pl.BlockSpec((_BM, D), zp_map),
        ],
        out_shape=[
            jax.ShapeDtypeStruct((N, D), jnp.float32),
            jax.ShapeDtypeStruct((N, D), jnp.float32),
        ],
        scratch_shapes=[
            pltpu.VMEM((P, N, D), jnp.bfloat16),
            pltpu.VMEM((P, N, D), jnp.bfloat16),
            pltpu.VMEM((P, 1, D), jnp.float32),
        ],
    )(mps_d, mps_p, h_d, h_p, Wt, b4, a4, wfct, bfc3, att3)
    return (z_d, z_p)


# speedup vs baseline: 1.1539x; 1.0946x over previous
import jax
import jax.numpy as jnp
from jax.experimental import pallas as pl
from jax.experimental.pallas import tpu as pltpu

_BM = 512
_K = 4


def _body_factory(P, N, D, nb):
    nchunk_side = P * nb
    nchunk = 2 * nchunk_side
    inv_n = 1.0 / N

    def body(mps_d_ref, mps_p_ref, h_d_ref, h_p_ref, wt_ref, b_ref, a_ref,
             wfct_ref, bfc_ref, att_ref, zd_ref, zp_ref,
             ring, fts_scr, e_scr, st_scr, zstage_d, zstage_p,
             ring_sems, zd_sem, zp_sem):

        def chunk_copy(c, slot):
            side_p = c >= nchunk_side
            il = c - jnp.where(side_p, nchunk_side, 0)
            pq = il // nb
            ii = il - pq * nb

            @pl.when(jnp.logical_not(side_p))
            def _d():
                pltpu.make_async_copy(
                    mps_d_ref.at[pq, pl.ds(ii * _BM, _BM), :],
                    ring.at[slot], ring_sems.at[slot]).start()

            @pl.when(side_p)
            def _p():
                pltpu.make_async_copy(
                    mps_p_ref.at[pq, pl.ds(ii * _BM, _BM), :],
                    ring.at[slot], ring_sems.at[slot]).start()

        def chunk_wait(c, slot):
            side_p = c >= nchunk_side
            il = c - jnp.where(side_p, nchunk_side, 0)
            pq = il // nb
            ii = il - pq * nb

            @pl.when(jnp.logical_not(side_p))
            def _d():
                pltpu.make_async_copy(
                    mps_d_ref.at[pq, pl.ds(ii * _BM, _BM), :],
                    ring.at[slot], ring_sems.at[slot]).wait()

            @pl.when(side_p)
            def _p():
                pltpu.make_async_copy(
                    mps_p_ref.at[pq, pl.ds(ii * _BM, _BM), :],
                    ring.at[slot], ring_sems.at[slot]).wait()

        def combine(side, zstage):
            att = att_ref[side]
            ls = [jnp.sum(st_scr[q] * att, keepdims=True) * inv_n
                  for q in range(P)]
            m = ls[0]
            for q in range(1, P):
                m = jnp.maximum(m, ls[q])
            ws = [jnp.exp(l - m) for l in ls]
            den = ws[0]
            for q in range(1, P):
                den = den + ws[q]
            for q in range(P):
                beta = ws[q] / den
                if q == 0:
                    zstage[...] = e_scr[q].astype(jnp.float32) * beta
                else:
                    zstage[...] = zstage[...] + e_scr[q].astype(jnp.float32) * beta

        for c0 in range(_K - 1):
            chunk_copy(jnp.int32(c0), c0)

        def step(c, _):
            slot = jax.lax.rem(c, _K)

            @pl.when(c + _K - 1 < nchunk)
            def _issue():
                chunk_copy(c + _K - 1, jax.lax.rem(c + _K - 1, _K))

            chunk_wait(c, slot)

            side_p = c >= nchunk_side
            il = c - jnp.where(side_p, nchunk_side, 0)
            pq = il // nb
            ii = il - pq * nb

            for s_static in range(2):
                for q in range(P):
                    @pl.when(c == s_static * nchunk_side + q)
                    def _fts(s_static=s_static, q=q):
                        h = (h_d_ref if s_static == 0 else h_p_ref)[...]
                        fts = jnp.dot(h.astype(jnp.bfloat16),
                                      wt_ref[s_static * P + q],
                                      preferred_element_type=jnp.float32)
                        fts_scr[q] = fts.astype(jnp.bfloat16)

            sp = jnp.where(side_p, P, 0) + pq
            adj = ring[slot].astype(jnp.bfloat16)
            acc = jnp.dot(adj, fts_scr[pq], preferred_element_type=jnp.float32)
            out = acc + b_ref[sp]
            out = jnp.where(out >= 0, out, a_ref[sp] * out)
            e_scr[pq, pl.ds(ii * _BM, _BM), :] = out.astype(jnp.bfloat16)
            sdx = jnp.where(side_p, 1, 0)
            pre = jnp.dot(out.astype(jnp.bfloat16), wfct_ref[sdx],
                          preferred_element_type=jnp.float32) + bfc_ref[sdx]
            col = jnp.sum(jnp.tanh(pre), axis=0, keepdims=True)

            @pl.when(ii == 0)
            def _init():
                st_scr[pq] = col

            @pl.when(ii > 0)
            def _acc():
                st_scr[pq] = st_scr[pq] + col

            @pl.when(c == nchunk_side - 1)
            def _zd():
                combine(0, zstage_d)
                pltpu.make_async_copy(zstage_d, zd_ref, zd_sem).start()

            @pl.when(c == nchunk - 1)
            def _zp():
                combine(1, zstage_p)
                pltpu.make_async_copy(zstage_p, zp_ref, zp_sem).start()

            return ()

        jax.lax.fori_loop(0, nchunk, step, ())
        pltpu.make_async_copy(zstage_d, zd_ref, zd_sem).wait()
        pltpu.make_async_copy(zstage_p, zp_ref, zp_sem).wait()

    return body


def kernel(h_d, h_p, mps_d, mps_p, W_dg, b_dg, a_dg, W_pt, b_pt, a_pt,
           Wfc_d, bfc_d, att_d, Wfc_p, bfc_p, att_p):
    P, N, _ = mps_d.shape
    D = h_d.shape[1]
    nb = N // _BM

    Wt = jnp.concatenate([jnp.transpose(W_dg, (0, 2, 1)),
                          jnp.transpose(W_pt, (0, 2, 1))]).astype(jnp.bfloat16)
    b2 = jnp.concatenate([b_dg, b_pt]).reshape(2 * P, 1, D)
    a2 = jnp.broadcast_to(
        jnp.concatenate([a_dg, a_pt]).reshape(2 * P, 1, 1), (2 * P, 1, D))
    wfct = jnp.stack([Wfc_d.T, Wfc_p.T]).astype(jnp.bfloat16)
    bfc3 = jnp.stack([bfc_d, bfc_p]).reshape(2, 1, D)
    att3 = jnp.stack([att_d, att_p])

    vm = pltpu.VMEM
    z_d, z_p = pl.pallas_call(
        _body_factory(P, N, D, nb),
        in_specs=[
            pl.BlockSpec(memory_space=pl.ANY),
            pl.BlockSpec(memory_space=pl.ANY),
            pl.BlockSpec(memory_space=vm),
            pl.BlockSpec(memory_space=vm),
            pl.BlockSpec(memory_space=vm),
            pl.BlockSpec(memory_space=vm),
            pl.BlockSpec(memory_space=vm),
            pl.BlockSpec(memory_space=vm),
            pl.BlockSpec(memory_space=vm),
            pl.BlockSpec(memory_space=vm),
        ],
        out_specs=[
            pl.BlockSpec(memory_space=pl.ANY),
            pl.BlockSpec(memory_space=pl.ANY),
        ],
        out_shape=[
            jax.ShapeDtypeStruct((N, D), jnp.float32),
            jax.ShapeDtypeStruct((N, D), jnp.float32),
        ],
        scratch_shapes=[
            pltpu.VMEM((_K, _BM, N), jnp.float32),
            pltpu.VMEM((P, N, D), jnp.bfloat16),
            pltpu.VMEM((P, N, D), jnp.bfloat16),
            pltpu.VMEM((P, 1, D), jnp.float32),
            pltpu.VMEM((N, D), jnp.float32),
            pltpu.VMEM((N, D), jnp.float32),
            pltpu.SemaphoreType.DMA((_K,)),
            pltpu.SemaphoreType.DMA,
            pltpu.SemaphoreType.DMA,
        ],
    )(mps_d, mps_p, h_d, h_p, Wt, b2, a2, wfct, bfc3, att3)
    return (z_d, z_p)
